# baseline (speedup 1.0000x reference)
import jax
import jax.numpy as jnp
from jax.experimental import pallas as pl
from jax.experimental.pallas import tpu as pltpu


def _cp():
    return pltpu.CompilerParams(
        dimension_semantics=("parallel",),
        vmem_limit_bytes=100 * 1024 * 1024,
    )


def _pick_tile(b: int, want: int) -> int:
    t = want
    while b % t:
        t //= 2
    return t


_S1 = (0, 1, 11, 12)
_S2 = (0, 1, 11, 12)
_S3 = (0, 1, 2, 11, 12, 13, 22, 23, 24)

_ROWS = 128


def _convs(x1, w1, b1, w2, b2, w3, b3, tb):
    rows = x1.shape[0]
    blk = tb * _ROWS
    r1 = blk - 12
    r2 = blk - 24
    r3 = blk - 48

    def body(x_ref, w1_ref, b1_ref, w2_ref, b2_ref, w3_ref, b3_ref, o_ref):
        xc1 = jnp.concatenate([x_ref[s:s + r1, :] for s in _S1], axis=1)
        a1 = jnp.dot(xc1, w1_ref[...], preferred_element_type=jnp.float32)
        a1 = jnp.maximum(a1 + b1_ref[...], 0.0).astype(jnp.bfloat16)
        xc2 = jnp.concatenate([a1[s:s + r2, :] for s in _S2], axis=1)
        a2 = jnp.dot(xc2, w2_ref[...], preferred_element_type=jnp.float32)
        a2 = jnp.maximum(a2 + b2_ref[...], 0.0).astype(jnp.bfloat16)
        gc = jnp.concatenate([a2[s:s + r3, :] for s in _S3], axis=1)
        a3 = jnp.dot(gc, w3_ref[...], preferred_element_type=jnp.float32)
        o_ref[0:r3, :] = jnp.maximum(a3 + b3_ref[...], 0.0).astype(o_ref.dtype)

    return pl.pallas_call(
        body,
        out_shape=jax.ShapeDtypeStruct((rows, 64), jnp.bfloat16),
        grid=(rows // blk,),
        in_specs=[
            pl.BlockSpec((blk, 256), lambda i: (i, 0)),
            pl.BlockSpec((1024, 128), lambda i: (0, 0)),
            pl.BlockSpec((1, 128), lambda i: (0, 0)),
            pl.BlockSpec((512, 64), lambda i: (0, 0)),
            pl.BlockSpec((1, 64), lambda i: (0, 0)),
            pl.BlockSpec((576, 64), lambda i: (0, 0)),
            pl.BlockSpec((1, 64), lambda i: (0, 0)),
        ],
        out_specs=pl.BlockSpec((blk, 64), lambda i: (i, 0)),
        compiler_params=_cp(),
    )(x1, w1, b1, w2, b2, w3, b3)


def _heads(feats, q1w, q1b, q2p, q2pb, i1w, i1b, i2p, i2pb, tb, a):
    b = feats.shape[0]
    bf = jnp.bfloat16

    def body(f_ref, q1w_ref, q1b_ref, q2_ref, q2b_ref,
             i1w_ref, i1b_ref, i2_ref, i2b_ref, q_ref, lsm_ref, i_ref):
        f = f_ref[...]
        hq = jnp.dot(f, q1w_ref[...].astype(bf), preferred_element_type=jnp.float32)
        hq = jnp.maximum(hq + q1b_ref[...], 0.0).astype(bf)
        q = jnp.dot(hq, q2_ref[...].astype(bf), preferred_element_type=jnp.float32)
        q_ref[...] = (q + q2b_ref[...])[:, :a]

        hi = jnp.dot(f, i1w_ref[...].astype(bf), preferred_element_type=jnp.float32)
        hi = jnp.maximum(hi + i1b_ref[...], 0.0).astype(bf)
        iv = jnp.dot(hi, i2_ref[...].astype(bf), preferred_element_type=jnp.float32)
        iv = iv + i2b_ref[...]
        i_ref[...] = iv[:, :a]

        col = jax.lax.broadcasted_iota(jnp.int32, iv.shape, 1)
        valid = col < a
        m = jnp.max(jnp.where(valid, iv, -jnp.inf), axis=-1, keepdims=True)
        s = iv - m
        e = jnp.where(valid, jnp.exp(s), 0.0)
        lsm = s - jnp.log(jnp.sum(e, axis=-1, keepdims=True))
        lsm_ref[...] = lsm[:, :a]

    res = lambda r, c: pl.BlockSpec((r, c), lambda i: (0, 0))
    row = lambda c: pl.BlockSpec((tb, c), lambda i: (i, 0))
    return pl.pallas_call(
        body,
        out_shape=(
            jax.ShapeDtypeStruct((b, a), jnp.float32),
            jax.ShapeDtypeStruct((b, a), jnp.float32),
            jax.ShapeDtypeStruct((b, a), jnp.float32),
        ),
        grid=(b // tb,),
        in_specs=[
            row(3136),
            res(3136, 512), res(1, 512), res(512, 128), res(1, 128),
            res(3136, 512), res(1, 512), res(512, 128), res(1, 128),
        ],
        out_specs=(row(a), row(a), row(a)),
        compiler_params=_cp(),
    )(feats, q1w, q1b, q2p, q2pb, i1w, i1b, i2p, i2pb)


def kernel(state, c1_w, c1_b, c2_w, c2_b, c3_w, c3_b,
           q1_w, q1_b, q2_w, q2_b, i1_w, i1_b, i2_w, i2_b):
    B = state.shape[0]
    A = q2_w.shape[1]
    bf = jnp.bfloat16

    xp = jnp.pad(state, ((0, 0), (0, 0), (0, 4), (0, 4)))
    xb = xp.reshape(B, 4, 11, 8, 11, 8).transpose(0, 2, 4, 3, 5, 1)
    xb = xb.reshape(B, 121, 256)
    x1 = jnp.pad(xb, ((0, 0), (0, _ROWS - 121), (0, 0))).reshape(B * _ROWS, 256)

    parts = []
    for pr in (0, 1):
        for qr in (0, 1):
            wp = jnp.pad(c1_w, ((4 * pr, 8 - 4 * pr), (4 * qr, 8 - 4 * qr),
                                (0, 0), (0, 0)))
            wp = wp.reshape(2, 8, 2, 8, 4, 32).transpose(0, 2, 1, 3, 4, 5)
            parts.append(wp.reshape(4, 256, 32))
    w1 = jnp.concatenate(parts, axis=-1).reshape(1024, 128).astype(bf)
    b1 = jnp.tile(c1_b, (1, 4))

    w2 = c2_w.reshape(2, 2, 2, 2, 32, 64).transpose(0, 2, 1, 3, 4, 5)
    w2 = w2.reshape(512, 64).astype(bf)
    w3 = c3_w.reshape(576, 64).astype(bf)

    tb = _pick_tile(B, 16)
    z = _convs(x1, w1, b1, w2, c2_b, w3, c3_b, tb)

    z = z.reshape(B, _ROWS, 64)[:, :121].reshape(B, 11, 11, 64)[:, :7, :7]
    feats = z.transpose(0, 3, 1, 2).reshape(B, 3136)

    pad_a = ((0, 0), (0, 128 - A))
    q2p, q2pb = jnp.pad(q2_w, pad_a), jnp.pad(q2_b, pad_a)
    i2p, i2pb = jnp.pad(i2_w, pad_a), jnp.pad(i2_b, pad_a)

    tbh = _pick_tile(B, 128)
    return _heads(feats, q1_w, q1_b, q2p, q2pb,
                  i1_w, i1_b, i2p, i2pb, tbh, A)


# speedup vs baseline: 1.1441x; 1.1441x over previous
import jax
import jax.numpy as jnp
from jax.experimental import pallas as pl
from jax.experimental.pallas import tpu as pltpu


def _cp():
    return pltpu.CompilerParams(
        dimension_semantics=("parallel",),
        vmem_limit_bytes=100 * 1024 * 1024,
    )


def _pick_tile(b: int, want: int) -> int:
    t = want
    while b % t:
        t //= 2
    return t


_S1 = (0, 1, 11, 12)
_S2 = (0, 1, 11, 12)
_S3 = (0, 1, 2, 11, 12, 13, 22, 23, 24)

_ROWS = 128


def _convs(x1, w1, b1, w2, b2, w3, b3, tb):
    rows = x1.shape[0]
    blk = tb * _ROWS
    r1 = blk - 12
    r2 = blk - 24
    r3 = blk - 48

    def body(x_ref, w1_ref, b1_ref, w2_ref, b2_ref, w3_ref, b3_ref, o_ref):
        xc1 = jnp.concatenate([x_ref[s:s + r1, :] for s in _S1], axis=1)
        a1 = jnp.dot(xc1, w1_ref[...], preferred_element_type=jnp.float32)
        a1 = jnp.maximum(a1 + b1_ref[...], 0.0).astype(jnp.bfloat16)
        xc2 = jnp.concatenate([a1[s:s + r2, :] for s in _S2], axis=1)
        a2 = jnp.dot(xc2, w2_ref[...], preferred_element_type=jnp.float32)
        a2 = jnp.maximum(a2 + b2_ref[...], 0.0).astype(jnp.bfloat16)
        gc = jnp.concatenate([a2[s:s + r3, :] for s in _S3], axis=1)
        a3 = jnp.dot(gc, w3_ref[...], preferred_element_type=jnp.float32)
        o_ref[0:r3, :] = jnp.maximum(a3 + b3_ref[...], 0.0).astype(o_ref.dtype)

    return pl.pallas_call(
        body,
        out_shape=jax.ShapeDtypeStruct((rows, 64), jnp.bfloat16),
        grid=(rows // blk,),
        in_specs=[
            pl.BlockSpec((blk, 256), lambda i: (i, 0)),
            pl.BlockSpec((1024, 128), lambda i: (0, 0)),
            pl.BlockSpec((1, 128), lambda i: (0, 0)),
            pl.BlockSpec((512, 64), lambda i: (0, 0)),
            pl.BlockSpec((1, 64), lambda i: (0, 0)),
            pl.BlockSpec((576, 64), lambda i: (0, 0)),
            pl.BlockSpec((1, 64), lambda i: (0, 0)),
        ],
        out_specs=pl.BlockSpec((blk, 64), lambda i: (i, 0)),
        compiler_params=_cp(),
    )(x1, w1, b1, w2, b2, w3, b3)


def _heads(feats, q1w, q1b, q2p, q2pb, i1w, i1b, i2p, i2pb, tb, a):
    b = feats.shape[0]
    bf = jnp.bfloat16

    def body(f_ref, q1w_ref, q1b_ref, q2_ref, q2b_ref,
             i1w_ref, i1b_ref, i2_ref, i2b_ref, q_ref, lsm_ref, i_ref):
        f = f_ref[...]
        hq = jnp.dot(f, q1w_ref[...].astype(bf), preferred_element_type=jnp.float32)
        hq = jnp.maximum(hq + q1b_ref[...], 0.0).astype(bf)
        q = jnp.dot(hq, q2_ref[...].astype(bf), preferred_element_type=jnp.float32)
        q_ref[...] = (q + q2b_ref[...])[:, :a]

        hi = jnp.dot(f, i1w_ref[...].astype(bf), preferred_element_type=jnp.float32)
        hi = jnp.maximum(hi + i1b_ref[...], 0.0).astype(bf)
        iv = jnp.dot(hi, i2_ref[...].astype(bf), preferred_element_type=jnp.float32)
        iv = iv + i2b_ref[...]
        i_ref[...] = iv[:, :a]

        col = jax.lax.broadcasted_iota(jnp.int32, iv.shape, 1)
        valid = col < a
        m = jnp.max(jnp.where(valid, iv, -jnp.inf), axis=-1, keepdims=True)
        s = iv - m
        e = jnp.where(valid, jnp.exp(s), 0.0)
        lsm = s - jnp.log(jnp.sum(e, axis=-1, keepdims=True))
        lsm_ref[...] = lsm[:, :a]

    res = lambda r, c: pl.BlockSpec((r, c), lambda i: (0, 0))
    row = lambda c: pl.BlockSpec((tb, c), lambda i: (i, 0))
    return pl.pallas_call(
        body,
        out_shape=(
            jax.ShapeDtypeStruct((b, a), jnp.float32),
            jax.ShapeDtypeStruct((b, a), jnp.float32),
            jax.ShapeDtypeStruct((b, a), jnp.float32),
        ),
        grid=(b // tb,),
        in_specs=[
            row(3136),
            res(3136, 512), res(1, 512), res(512, 128), res(1, 128),
            res(3136, 512), res(1, 512), res(512, 128), res(1, 128),
        ],
        out_specs=(row(a), row(a), row(a)),
        compiler_params=_cp(),
    )(feats, q1w, q1b, q2p, q2pb, i1w, i1b, i2p, i2pb)


def kernel(state, c1_w, c1_b, c2_w, c2_b, c3_w, c3_b,
           q1_w, q1_b, q2_w, q2_b, i1_w, i1_b, i2_w, i2_b):
    B = state.shape[0]
    A = q2_w.shape[1]
    bf = jnp.bfloat16

    xp = jnp.pad(state.astype(bf), ((0, 0), (0, 0), (0, 4), (0, 4)))
    xb = xp.reshape(B, 4, 11, 8, 11, 8).transpose(0, 2, 4, 3, 5, 1)
    xb = xb.reshape(B, 121, 256)
    x1 = jnp.pad(xb, ((0, 0), (0, _ROWS - 121), (0, 0))).reshape(B * _ROWS, 256)

    parts = []
    for pr in (0, 1):
        for qr in (0, 1):
            wp = jnp.pad(c1_w, ((4 * pr, 8 - 4 * pr), (4 * qr, 8 - 4 * qr),
                                (0, 0), (0, 0)))
            wp = wp.reshape(2, 8, 2, 8, 4, 32).transpose(0, 2, 1, 3, 4, 5)
            parts.append(wp.reshape(4, 256, 32))
    w1 = jnp.concatenate(parts, axis=-1).reshape(1024, 128).astype(bf)
    b1 = jnp.tile(c1_b, (1, 4))

    w2 = c2_w.reshape(2, 2, 2, 2, 32, 64).transpose(0, 2, 1, 3, 4, 5)
    w2 = w2.reshape(512, 64).astype(bf)
    w3 = c3_w.reshape(576, 64).astype(bf)

    tb = _pick_tile(B, 16)
    z = _convs(x1, w1, b1, w2, c2_b, w3, c3_b, tb)

    z = z.reshape(B, _ROWS, 64)[:, :121].reshape(B, 11, 11, 64)[:, :7, :7]
    feats = z.transpose(0, 3, 1, 2).reshape(B, 3136)

    pad_a = ((0, 0), (0, 128 - A))
    q2p, q2pb = jnp.pad(q2_w, pad_a), jnp.pad(q2_b, pad_a)
    i2p, i2pb = jnp.pad(i2_w, pad_a), jnp.pad(i2_b, pad_a)

    tbh = _pick_tile(B, 128)
    return _heads(feats, q1_w, q1_b, q2p, q2pb,
                  i1_w, i1_b, i2p, i2pb, tbh, A)


# speedup vs baseline: 1.1534x; 1.0082x over previous
import jax
import jax.numpy as jnp
from jax.experimental import pallas as pl
from jax.experimental.pallas import tpu as pltpu


def _cp():
    return pltpu.CompilerParams(
        dimension_semantics=("parallel",),
        vmem_limit_bytes=100 * 1024 * 1024,
    )


def _pick_tile(b: int, want: int) -> int:
    t = want
    while b % t:
        t //= 2
    return t


_S1 = (0, 1, 11, 12)
_S2 = (0, 1, 11, 12)
_S3 = (0, 1, 2, 11, 12, 13, 22, 23, 24)

_ROWS = 128


def _convs(x1, w1, b1, w2, b2, w3, b3, tb):
    rows = x1.shape[0]
    blk = tb * _ROWS
    r1 = blk - 12
    r2 = blk - 24
    r3 = blk - 48

    def body(x_ref, w1_ref, b1_ref, w2_ref, b2_ref, w3_ref, b3_ref, o_ref):
        xc1 = jnp.concatenate([x_ref[s:s + r1, :] for s in _S1], axis=1)
        a1 = jnp.dot(xc1, w1_ref[...], preferred_element_type=jnp.float32)
        a1 = jnp.maximum(a1 + b1_ref[...], 0.0).astype(jnp.bfloat16)
        xc2 = jnp.concatenate([a1[s:s + r2, :] for s in _S2], axis=1)
        a2 = jnp.dot(xc2, w2_ref[...], preferred_element_type=jnp.float32)
        a2 = jnp.maximum(a2 + b2_ref[...], 0.0).astype(jnp.bfloat16)
        gc = jnp.concatenate([a2[s:s + r3, :] for s in _S3], axis=1)
        a3 = jnp.dot(gc, w3_ref[...], preferred_element_type=jnp.float32)
        o_ref[0:r3, :] = jnp.maximum(a3 + b3_ref[...], 0.0).astype(o_ref.dtype)

    return pl.pallas_call(
        body,
        out_shape=jax.ShapeDtypeStruct((rows, 64), jnp.bfloat16),
        grid=(rows // blk,),
        in_specs=[
            pl.BlockSpec((blk, 256), lambda i: (i, 0)),
            pl.BlockSpec((1024, 128), lambda i: (0, 0)),
            pl.BlockSpec((1, 128), lambda i: (0, 0)),
            pl.BlockSpec((512, 64), lambda i: (0, 0)),
            pl.BlockSpec((1, 64), lambda i: (0, 0)),
            pl.BlockSpec((576, 64), lambda i: (0, 0)),
            pl.BlockSpec((1, 64), lambda i: (0, 0)),
        ],
        out_specs=pl.BlockSpec((blk, 64), lambda i: (i, 0)),
        compiler_params=_cp(),
    )(x1, w1, b1, w2, b2, w3, b3)


def _heads(feats, q1w, q1b, q2p, q2pb, i1w, i1b, i2p, i2pb, tb, a):
    b = feats.shape[0]
    bf = jnp.bfloat16

    def body(f_ref, q1w_ref, q1b_ref, q2_ref, q2b_ref,
             i1w_ref, i1b_ref, i2_ref, i2b_ref, q_ref, lsm_ref, i_ref):
        f = f_ref[...]
        hq = jnp.dot(f, q1w_ref[...].astype(bf), preferred_element_type=jnp.float32)
        hq = jnp.maximum(hq + q1b_ref[...], 0.0).astype(bf)
        q = jnp.dot(hq, q2_ref[...].astype(bf), preferred_element_type=jnp.float32)
        q_ref[...] = (q + q2b_ref[...])[:, :a]

        hi = jnp.dot(f, i1w_ref[...].astype(bf), preferred_element_type=jnp.float32)
        hi = jnp.maximum(hi + i1b_ref[...], 0.0).astype(bf)
        iv = jnp.dot(hi, i2_ref[...].astype(bf), preferred_element_type=jnp.float32)
        iv = iv + i2b_ref[...]
        i_ref[...] = iv[:, :a]

        col = jax.lax.broadcasted_iota(jnp.int32, iv.shape, 1)
        valid = col < a
        m = jnp.max(jnp.where(valid, iv, -jnp.inf), axis=-1, keepdims=True)
        s = iv - m
        e = jnp.where(valid, jnp.exp(s), 0.0)
        lsm = s - jnp.log(jnp.sum(e, axis=-1, keepdims=True))
        lsm_ref[...] = lsm[:, :a]

    res = lambda r, c: pl.BlockSpec((r, c), lambda i: (0, 0))
    row = lambda c: pl.BlockSpec((tb, c), lambda i: (i, 0))
    return pl.pallas_call(
        body,
        out_shape=(
            jax.ShapeDtypeStruct((b, a), jnp.float32),
            jax.ShapeDtypeStruct((b, a), jnp.float32),
            jax.ShapeDtypeStruct((b, a), jnp.float32),
        ),
        grid=(b // tb,),
        in_specs=[
            row(3136),
            res(3136, 512), res(1, 512), res(512, 128), res(1, 128),
            res(3136, 512), res(1, 512), res(512, 128), res(1, 128),
        ],
        out_specs=(row(a), row(a), row(a)),
        compiler_params=_cp(),
    )(feats, q1w, q1b, q2p, q2pb, i1w, i1b, i2p, i2pb)


def kernel(state, c1_w, c1_b, c2_w, c2_b, c3_w, c3_b,
           q1_w, q1_b, q2_w, q2_b, i1_w, i1_b, i2_w, i2_b):
    B = state.shape[0]
    A = q2_w.shape[1]
    bf = jnp.bfloat16

    xp = jnp.pad(state.astype(bf), ((0, 0), (0, 0), (0, 4), (0, 4)))
    xb = xp.reshape(B, 4, 11, 8, 11, 8).transpose(0, 2, 4, 3, 5, 1)
    xb = xb.reshape(B, 121, 256)
    x1 = jnp.pad(xb, ((0, 0), (0, _ROWS - 121), (0, 0))).reshape(B * _ROWS, 256)

    parts = []
    for pr in (0, 1):
        for qr in (0, 1):
            wp = jnp.pad(c1_w, ((4 * pr, 8 - 4 * pr), (4 * qr, 8 - 4 * qr),
                                (0, 0), (0, 0)))
            wp = wp.reshape(2, 8, 2, 8, 4, 32).transpose(0, 2, 1, 3, 4, 5)
            parts.append(wp.reshape(4, 256, 32))
    w1 = jnp.concatenate(parts, axis=-1).reshape(1024, 128).astype(bf)
    b1 = jnp.tile(c1_b, (1, 4))

    w2 = c2_w.reshape(2, 2, 2, 2, 32, 64).transpose(0, 2, 1, 3, 4, 5)
    w2 = w2.reshape(512, 64).astype(bf)
    w3 = c3_w.reshape(576, 64).astype(bf)

    tb = _pick_tile(B, 32)
    z = _convs(x1, w1, b1, w2, c2_b, w3, c3_b, tb)

    z = z.reshape(B, _ROWS, 64)[:, :121].reshape(B, 11, 11, 64)[:, :7, :7]
    feats = z.transpose(0, 3, 1, 2).reshape(B, 3136)

    pad_a = ((0, 0), (0, 128 - A))
    q2p, q2pb = jnp.pad(q2_w, pad_a), jnp.pad(q2_b, pad_a)
    i2p, i2pb = jnp.pad(i2_w, pad_a), jnp.pad(i2_b, pad_a)

    tbh = _pick_tile(B, 128)
    return _heads(feats, q1_w, q1_b, q2p, q2pb,
                  i1_w, i1_b, i2p, i2pb, tbh, A)
